# baseline (speedup 1.0000x reference)
import jax
import jax.numpy as jnp
from jax import lax
from jax.experimental import pallas as pl
from jax.experimental.pallas import tpu as pltpu

H = 32
W = 32
HALF = 128
EMBED = 2 * HALF


def _make_pos_kernel(b):
    def _pos_kernel(rows_ref, cols_ref, out_ref, scratch, sem):
        cols_t = jnp.transpose(cols_ref[0:W, :], (1, 0))
        rows_t = jnp.transpose(rows_ref[0:H, :], (1, 0))
        top = jnp.broadcast_to(cols_t[:, None, :], (HALF, H, W))
        bot = jnp.broadcast_to(rows_t[:, :, None], (HALF, H, W))
        scratch[0:HALF] = top
        scratch[HALF:EMBED] = bot

        for i in range(b):
            pltpu.make_async_copy(scratch, out_ref.at[i], sem).start()
        for i in range(b):
            pltpu.make_async_copy(scratch, out_ref.at[i], sem).wait()

    return _pos_kernel


def kernel(pixel_values, rows_emb, cols_emb):
    b = pixel_values.shape[0]
    return pl.pallas_call(
        _make_pos_kernel(b),
        in_specs=[
            pl.BlockSpec(memory_space=pltpu.VMEM),
            pl.BlockSpec(memory_space=pltpu.VMEM),
        ],
        out_specs=pl.BlockSpec(memory_space=pl.ANY),
        out_shape=jax.ShapeDtypeStruct((b, EMBED, H, W), jnp.float32),
        scratch_shapes=[
            pltpu.VMEM((EMBED, H, W), jnp.float32),
            pltpu.SemaphoreType.DMA,
        ],
    )(rows_emb, cols_emb)


# speedup vs baseline: 10.5073x; 10.5073x over previous
import jax
import jax.numpy as jnp
from jax import lax
from jax.experimental import pallas as pl
from jax.experimental.pallas import tpu as pltpu

H = 32
W = 32
HALF = 128
EMBED = 2 * HALF
P = H * W


def _make_pos_kernel(b):
    def _pos_kernel(rows_ref, cols_ref, out_ref, scratch, sem):
        p_idx = lax.broadcasted_iota(jnp.int32, (P, W), 0)
        q_idx = lax.broadcasted_iota(jnp.int32, (P, W), 1)
        sel_c = (p_idx % W == q_idx).astype(jnp.float32)
        sel_r = (p_idx // W == q_idx).astype(jnp.float32)
        dn = (((1,), (0,)), ((), ()))
        top = lax.dot_general(sel_c, cols_ref[0:W, :], dn,
                              preferred_element_type=jnp.float32,
                              precision=lax.Precision.HIGHEST)
        bot = lax.dot_general(sel_r, rows_ref[0:H, :], dn,
                              preferred_element_type=jnp.float32,
                              precision=lax.Precision.HIGHEST)
        scratch[:, 0:HALF] = top
        scratch[:, HALF:EMBED] = bot

        for i in range(b):
            pltpu.make_async_copy(scratch, out_ref.at[i], sem).start()
        for i in range(b):
            pltpu.make_async_copy(scratch, out_ref.at[i], sem).wait()

    return _pos_kernel


def kernel(pixel_values, rows_emb, cols_emb):
    b = pixel_values.shape[0]
    out = pl.pallas_call(
        _make_pos_kernel(b),
        in_specs=[
            pl.BlockSpec(memory_space=pltpu.VMEM),
            pl.BlockSpec(memory_space=pltpu.VMEM),
        ],
        out_specs=pl.BlockSpec(memory_space=pl.ANY),
        out_shape=jax.ShapeDtypeStruct((b, P, EMBED), jnp.float32),
        scratch_shapes=[
            pltpu.VMEM((P, EMBED), jnp.float32),
            pltpu.SemaphoreType.DMA,
        ],
    )(rows_emb, cols_emb)
    return out.reshape(b, H, W, EMBED).transpose(0, 3, 1, 2)


# speedup vs baseline: 10.9187x; 1.0392x over previous
import jax
import jax.numpy as jnp
from jax import lax
from jax.experimental import pallas as pl
from jax.experimental.pallas import tpu as pltpu

H = 32
W = 32
HALF = 128
EMBED = 2 * HALF
P = H * W
NCHUNK = 4
CP = P // NCHUNK


def _make_pos_kernel(b):
    def _pos_kernel(rows_ref, cols_ref, out_ref, scratch, sem):
        dn = (((1,), (0,)), ((), ()))
        for k in range(NCHUNK):
            p_idx = lax.broadcasted_iota(jnp.int32, (CP, W), 0) + k * CP
            q_idx = lax.broadcasted_iota(jnp.int32, (CP, W), 1)
            sel_c = (p_idx % W == q_idx).astype(jnp.float32)
            sel_r = (p_idx // W == q_idx).astype(jnp.float32)
            top = lax.dot_general(sel_c, cols_ref[0:W, :], dn,
                                  preferred_element_type=jnp.float32,
                                  precision=lax.Precision.HIGHEST)
            bot = lax.dot_general(sel_r, rows_ref[0:H, :], dn,
                                  preferred_element_type=jnp.float32,
                                  precision=lax.Precision.HIGHEST)
            scratch[k * CP:(k + 1) * CP, 0:HALF] = top
            scratch[k * CP:(k + 1) * CP, HALF:EMBED] = bot
            chunk = scratch.at[pl.ds(k * CP, CP), :]
            for i in range(b):
                pltpu.make_async_copy(
                    chunk, out_ref.at[i, pl.ds(k * CP, CP), :], sem).start()
        for k in range(NCHUNK):
            chunk = scratch.at[pl.ds(k * CP, CP), :]
            for i in range(b):
                pltpu.make_async_copy(
                    chunk, out_ref.at[i, pl.ds(k * CP, CP), :], sem).wait()

    return _pos_kernel


def kernel(pixel_values, rows_emb, cols_emb):
    b = pixel_values.shape[0]
    out = pl.pallas_call(
        _make_pos_kernel(b),
        in_specs=[
            pl.BlockSpec(memory_space=pltpu.VMEM),
            pl.BlockSpec(memory_space=pltpu.VMEM),
        ],
        out_specs=pl.BlockSpec(memory_space=pl.ANY),
        out_shape=jax.ShapeDtypeStruct((b, P, EMBED), jnp.float32),
        scratch_shapes=[
            pltpu.VMEM((P, EMBED), jnp.float32),
            pltpu.SemaphoreType.DMA,
        ],
    )(rows_emb, cols_emb)
    return out.reshape(b, H, W, EMBED).transpose(0, 3, 1, 2)


# speedup vs baseline: 10.9311x; 1.0011x over previous
import jax
import jax.numpy as jnp
from jax import lax
from jax.experimental import pallas as pl
from jax.experimental.pallas import tpu as pltpu

H = 32
W = 32
HALF = 128
EMBED = 2 * HALF
P = H * W
NCHUNK = 8
CP = P // NCHUNK


def _make_pos_kernel(b):
    def _pos_kernel(rows_ref, cols_ref, out_ref, scratch, sem):
        dn = (((1,), (0,)), ((), ()))
        for k in range(NCHUNK):
            p_idx = lax.broadcasted_iota(jnp.int32, (CP, W), 0) + k * CP
            q_idx = lax.broadcasted_iota(jnp.int32, (CP, W), 1)
            sel_c = (p_idx % W == q_idx).astype(jnp.float32)
            sel_r = (p_idx // W == q_idx).astype(jnp.float32)
            top = lax.dot_general(sel_c, cols_ref[0:W, :], dn,
                                  preferred_element_type=jnp.float32,
                                  precision=lax.Precision.HIGHEST)
            bot = lax.dot_general(sel_r, rows_ref[0:H, :], dn,
                                  preferred_element_type=jnp.float32,
                                  precision=lax.Precision.HIGHEST)
            scratch[k * CP:(k + 1) * CP, 0:HALF] = top
            scratch[k * CP:(k + 1) * CP, HALF:EMBED] = bot
            chunk = scratch.at[pl.ds(k * CP, CP), :]
            for i in range(b):
                pltpu.make_async_copy(
                    chunk, out_ref.at[i, pl.ds(k * CP, CP), :], sem).start()
        for k in range(NCHUNK):
            chunk = scratch.at[pl.ds(k * CP, CP), :]
            for i in range(b):
                pltpu.make_async_copy(
                    chunk, out_ref.at[i, pl.ds(k * CP, CP), :], sem).wait()

    return _pos_kernel


def kernel(pixel_values, rows_emb, cols_emb):
    b = pixel_values.shape[0]
    out = pl.pallas_call(
        _make_pos_kernel(b),
        in_specs=[
            pl.BlockSpec(memory_space=pltpu.VMEM),
            pl.BlockSpec(memory_space=pltpu.VMEM),
        ],
        out_specs=pl.BlockSpec(memory_space=pl.ANY),
        out_shape=jax.ShapeDtypeStruct((b, P, EMBED), jnp.float32),
        scratch_shapes=[
            pltpu.VMEM((P, EMBED), jnp.float32),
            pltpu.SemaphoreType.DMA,
        ],
    )(rows_emb, cols_emb)
    return out.reshape(b, H, W, EMBED).transpose(0, 3, 1, 2)
